# baseline (speedup 1.0000x reference)
import functools

import jax
import jax.numpy as jnp
from jax import lax
from jax.experimental import pallas as pl
from jax.experimental.pallas import tpu as pltpu
from jax.experimental.pallas import tpu_sc as plsc

B = 16384
D = 64
LANES = 16
NC, NS = 2, 16
NW = NC * NS
ROWS = B // NW
CHUNK = 128
NCHUNK = ROWS // CHUNK
MARGIN = 1.0
K = D // LANES


def _rsqrt(x):
    i = plsc.bitcast(x, jnp.int32)
    i = jnp.int32(0x5F3759DF) - lax.shift_right_logical(i, 1)
    y = plsc.bitcast(i, jnp.float32)
    for _ in range(3):
        y = y * (1.5 - 0.5 * x * y * y)
    return y


def _transfer_row(e_buf, t_buf, rtk, r):
    ek = [e_buf[r, pl.ds(16 * k, 16)] for k in range(K)]
    tk = [t_buf[r, pl.ds(16 * k, 16)] for k in range(K)]
    d = ek[0] * tk[0]
    for k in range(1, K):
        d = d + ek[k] * tk[k]
    dsum = jnp.full((LANES,), jnp.sum(d), jnp.float32)
    vk = [ek[k] + dsum * rtk[k] for k in range(K)]
    s2 = vk[0] * vk[0]
    for k in range(1, K):
        s2 = s2 + vk[k] * vk[k]
    s2s = jnp.full((LANES,), jnp.sum(s2), jnp.float32)
    y = _rsqrt(s2s)
    norm = s2s * y
    inv = 1.0 / jnp.maximum(norm, 1e-12)
    return [vk[k] * inv for k in range(K)]


def _side_score(heb, htb, teb, ttb, reb, rtb, r):
    rtk = [rtb[r, pl.ds(16 * k, 16)] for k in range(K)]
    hk = _transfer_row(heb, htb, rtk, r)
    tk = _transfer_row(teb, ttb, rtk, r)
    acc = None
    for k in range(K):
        rek = reb[r, pl.ds(16 * k, 16)]
        term = jnp.abs(hk[k] + rek - tk[k])
        acc = term if acc is None else acc + term
    return jnp.sum(acc)


def _disc_body(ph, pt, pr, nh, nt, nr, takef,
               ent_emb, rel_emb, ent_tr, rel_tr,
               nscore_out, partial_out,
               iph, ipt, ipr, inh, intt, inr,
               phe, pht, pte, ptt, pre, prt,
               nhe, nht, nte, ntt, nre, nrt,
               take_v, ns_buf, loss_buf, isem, gsem):
    wid = lax.axis_index("s") * NC + lax.axis_index("c")
    base = wid * ROWS
    lane = lax.iota(jnp.int32, LANES)

    pltpu.sync_copy(takef.at[pl.ds(base, ROWS)], take_v)

    lossv = jnp.zeros((LANES,), jnp.float32)
    for c in range(NCHUNK):
        off = base + c * CHUNK
        idx_cps = [
            pltpu.async_copy(src.at[pl.ds(off, CHUNK)], dst, isem)
            for src, dst in ((ph, iph), (pt, ipt), (pr, ipr),
                             (nh, inh), (nt, intt), (nr, inr))
        ]
        for cp in idx_cps:
            cp.wait()
        gathers = [
            pltpu.async_copy(tab.at[idx], dst, gsem)
            for tab, idx, dst in (
                (ent_emb, iph, phe), (ent_tr, iph, pht),
                (ent_emb, ipt, pte), (ent_tr, ipt, ptt),
                (rel_emb, ipr, pre), (rel_tr, ipr, prt),
                (ent_emb, inh, nhe), (ent_tr, inh, nht),
                (ent_emb, intt, nte), (ent_tr, intt, ntt),
                (rel_emb, inr, nre), (rel_tr, inr, nrt),
            )
        ]
        for cp in gathers:
            cp.wait()

        def group_body(g, lossv_c):
            def row_body(i, carry):
                nsv, psv = carry
                r = g * LANES + i
                p_s = _side_score(phe, pht, pte, ptt, pre, prt, r)
                n_s = _side_score(nhe, nht, nte, ntt, nre, nrt, r)
                onehot = lane == i
                nsv = jnp.where(onehot,
                                jnp.full((LANES,), -n_s, jnp.float32), nsv)
                psv = jnp.where(onehot,
                                jnp.full((LANES,), p_s, jnp.float32), psv)
                return nsv, psv

            z = jnp.zeros((LANES,), jnp.float32)
            nsv, psv = lax.fori_loop(0, LANES, row_body, (z, z))
            tkv = take_v[pl.ds(c * CHUNK + g * LANES, LANES)]
            lossv_c = lossv_c + jnp.maximum(0.0, psv + nsv + MARGIN) * tkv
            ns_buf[pl.ds(c * CHUNK + g * LANES, LANES)] = nsv
            return lossv_c

        lossv = lax.fori_loop(0, CHUNK // LANES, group_body, lossv)

    pltpu.sync_copy(ns_buf, nscore_out.at[pl.ds(base, ROWS)])
    loss_buf[...] = lossv
    pltpu.sync_copy(loss_buf, partial_out.at[wid])


_disc = functools.partial(
    pl.kernel,
    mesh=plsc.VectorSubcoreMesh(core_axis_name="c", subcore_axis_name="s"),
    compiler_params=pltpu.CompilerParams(needs_layout_passes=False,
                                         use_tc_tiling_on_sc=False),
    out_type=[
        jax.ShapeDtypeStruct((B,), jnp.float32),
        jax.ShapeDtypeStruct((NW, LANES), jnp.float32),
    ],
    scratch_types=(
        [pltpu.VMEM((CHUNK,), jnp.int32) for _ in range(6)]
        + [pltpu.VMEM((CHUNK, D), jnp.float32) for _ in range(12)]
        + [pltpu.VMEM((ROWS,), jnp.float32),
           pltpu.VMEM((ROWS,), jnp.float32),
           pltpu.VMEM((LANES,), jnp.float32),
           pltpu.SemaphoreType.DMA,
           pltpu.SemaphoreType.DMA]
    ),
)(_disc_body)


def _sum_body(p_ref, o_ref):
    o_ref[0, 0] = jnp.sum(p_ref[...])


_sum_partials = pl.pallas_call(
    _sum_body,
    out_shape=jax.ShapeDtypeStruct((1, 1), jnp.float32),
    out_specs=pl.BlockSpec(memory_space=pltpu.SMEM),
)


def kernel(pos_h, pos_r, pos_t, neg_h, neg_r, neg_t, take,
           ent_emb_w, rel_emb_w, ent_transfer_w, rel_transfer_w):
    ph = pos_h.astype(jnp.int32)
    pt = pos_t.astype(jnp.int32)
    pr = pos_r.astype(jnp.int32)
    nh = neg_h.astype(jnp.int32)
    nt = neg_t.astype(jnp.int32)
    nr = neg_r.astype(jnp.int32)
    takef = take.astype(jnp.float32)
    nscore, partials = _disc(ph, pt, pr, nh, nt, nr, takef,
                             ent_emb_w, rel_emb_w,
                             ent_transfer_w, rel_transfer_w)
    loss = _sum_partials(partials)[0, 0]
    return (loss, nscore)
